# Initial kernel scaffold; baseline (speedup 1.0000x reference)
import functools
import math

import jax
import jax.numpy as jnp
from jax import lax
from jax.experimental import pallas as pl
from jax.experimental.pallas import tpu as pltpu
from jax.experimental.pallas import tpu_sc as plsc

N = 10000
E = 320000
D_IN = 128
D_H = 128
D_OUT = 64

NP = 10240
CHUNK = 128
E_PAD = 323584
KA = E_PAD // (32 * CHUNK)
KC = E_PAD // (16 * CHUNK)
ROWS_PER_TILE = NP // 16

_mesh = plsc.VectorSubcoreMesh(core_axis_name="c", subcore_axis_name="s")


def _zero_rows(ref, nrows, width):
    zv = jnp.zeros((16,), jnp.float32)

    def body(i, _):
        for d in range(width // 16):
            ref[i, pl.ds(16 * d, 16)] = zv
        return 0

    lax.fori_loop(0, nrows, body, 0)


@functools.partial(
    pl.kernel,
    out_type=jax.ShapeDtypeStruct((2, NP, 16), jnp.float32),
    mesh=_mesh,
    scratch_types=[
        pltpu.VMEM((KA, CHUNK), jnp.int32),
        pltpu.VMEM((CHUNK, 16), jnp.float32),
        pltpu.VMEM((CHUNK, 16), jnp.float32),
        pltpu.VMEM_SHARED((NP, 16), jnp.float32),
    ],
)
def _deg_kernel(dst3_hbm, deg_hbm, dst_v, ones_v, zero_v, acc_s):
    cid = lax.axis_index("c")
    sid = lax.axis_index("s")
    wid = cid * 16 + sid

    pltpu.sync_copy(dst3_hbm.at[wid], dst_v)

    ov = jnp.full((16,), 1.0, jnp.float32)

    def fill(i, _):
        ones_v[i, pl.ds(0, 16)] = ov
        zero_v[i, pl.ds(0, 16)] = jnp.zeros((16,), jnp.float32)
        return 0

    lax.fori_loop(0, CHUNK, fill, 0)

    base = sid * ROWS_PER_TILE
    for k in range(ROWS_PER_TILE // CHUNK):
        pltpu.sync_copy(zero_v, acc_s.at[pl.ds(base + k * CHUNK, CHUNK)])
    plsc.subcore_barrier()

    def edge_chunk(j, _):
        pltpu.sync_copy(ones_v, acc_s.at[dst_v.at[j]], add=True)
        return 0

    lax.fori_loop(0, KA, edge_chunk, 0)
    plsc.subcore_barrier()

    pltpu.sync_copy(
        acc_s.at[pl.ds(base, ROWS_PER_TILE)],
        deg_hbm.at[cid, pl.ds(base, ROWS_PER_TILE)],
    )


def _make_edge_kernel(dw):
    @functools.partial(
        pl.kernel,
        out_type=(
            jax.ShapeDtypeStruct((NP, dw), jnp.float32),
            jax.ShapeDtypeStruct((NP, dw), jnp.float32),
        ),
        mesh=_mesh,
        scratch_types=[
            pltpu.VMEM((KC, CHUNK), jnp.int32),
            pltpu.VMEM((KC, CHUNK), jnp.int32),
            pltpu.VMEM((CHUNK, dw), jnp.float32),
            pltpu.VMEM_SHARED((NP, dw), jnp.float32),
            pltpu.SemaphoreType.DMA,
        ],
    )
    def edge_kernel(glo_hbm, ghi_hbm, src3_hbm, dst3_hbm,
                    alo_hbm, ahi_hbm, src_v, dst_v, rows_v, acc_s, sem):
        cid = lax.axis_index("c")
        sid = lax.axis_index("s")

        pltpu.sync_copy(src3_hbm.at[sid], src_v)
        pltpu.sync_copy(dst3_hbm.at[sid], dst_v)

        _zero_rows(rows_v, CHUNK, dw)
        base = sid * ROWS_PER_TILE
        for k in range(ROWS_PER_TILE // CHUNK):
            pltpu.sync_copy(rows_v, acc_s.at[pl.ds(base + k * CHUNK, CHUNK)])
        plsc.subcore_barrier()

        def chunk_lo(j, _):
            pltpu.async_copy(glo_hbm.at[src_v.at[j]], rows_v, sem).wait()
            pltpu.sync_copy(rows_v, acc_s.at[dst_v.at[j]], add=True)
            return 0

        def chunk_hi(j, _):
            pltpu.async_copy(ghi_hbm.at[src_v.at[j]], rows_v, sem).wait()
            pltpu.sync_copy(rows_v, acc_s.at[dst_v.at[j]], add=True)
            return 0

        @pl.when(cid == 0)
        def _():
            lax.fori_loop(0, KC, chunk_lo, 0)

        @pl.when(cid == 1)
        def _():
            lax.fori_loop(0, KC, chunk_hi, 0)

        plsc.subcore_barrier()

        @pl.when(cid == 0)
        def _():
            pltpu.sync_copy(acc_s.at[pl.ds(base, ROWS_PER_TILE)],
                            alo_hbm.at[pl.ds(base, ROWS_PER_TILE)])

        @pl.when(cid == 1)
        def _():
            pltpu.sync_copy(acc_s.at[pl.ds(base, ROWS_PER_TILE)],
                            ahi_hbm.at[pl.ds(base, ROWS_PER_TILE)])

    return edge_kernel


_edge_kernel_64 = _make_edge_kernel(64)
_edge_kernel_32 = _make_edge_kernel(32)


BLK = 1024
GRID = NP // BLK


def _dinv_blk(d0_ref, d1_ref):
    deg = d0_ref[:, 0] + d1_ref[:, 0] + 1.0
    return 1.0 / jnp.sqrt(deg)


def _tc_first_body(x_ref, w_ref, d0_ref, d1_ref, glo_ref, ghi_ref):
    dinv = _dinv_blk(d0_ref, d1_ref)
    h = jnp.dot(x_ref[...], w_ref[...], preferred_element_type=jnp.float32)
    g = h * dinv[:, None]
    glo_ref[...] = g[:, :64]
    ghi_ref[...] = g[:, 64:]


def _tc_mid_body(alo_ref, ahi_ref, glo_ref, ghi_ref, d0_ref, d1_ref,
                 w_ref, s_ref, t_ref, olo_ref, ohi_ref, *, dout):
    dinv = _dinv_blk(d0_ref, d1_ref)
    u = jnp.concatenate(
        [alo_ref[...] + glo_ref[...], ahi_ref[...] + ghi_ref[...]], axis=1)
    u = u * dinv[:, None]
    h = jnp.maximum(u * s_ref[...] + t_ref[...], 0.0)
    g = jnp.dot(h, w_ref[...], preferred_element_type=jnp.float32)
    g = g * dinv[:, None]
    olo_ref[...] = g[:, : dout // 2]
    ohi_ref[...] = g[:, dout // 2:]


def _tc_last_body(alo_ref, ahi_ref, glo_ref, ghi_ref, d0_ref, d1_ref,
                  b_ref, out_ref):
    dinv = _dinv_blk(d0_ref, d1_ref)
    z = jnp.concatenate(
        [alo_ref[...] + glo_ref[...], ahi_ref[...] + ghi_ref[...]], axis=1)
    z = z * dinv[:, None] + b_ref[...]
    m = jnp.max(z, axis=1, keepdims=True)
    zs = z - m
    out_ref[...] = zs - jnp.log(jnp.sum(jnp.exp(zs), axis=1, keepdims=True))


def _row_spec(w):
    return pl.BlockSpec((BLK, w), lambda i: (i, 0))


def _full_spec(shape):
    return pl.BlockSpec(shape, lambda i: tuple(0 for _ in shape))


def kernel(x, edge_index, W1, b1, g1, be1, W2, b2, g2, be2, W3, b3):
    f32 = jnp.float32
    src = edge_index[0].astype(jnp.int32)
    dst = edge_index[1].astype(jnp.int32)
    pad = jnp.full((E_PAD - E,), N, jnp.int32)
    srcp = jnp.concatenate([src, pad])
    dstp = jnp.concatenate([dst, pad])
    dst_a = dstp.reshape(32, KA, CHUNK)
    src_c = srcp.reshape(16, KC, CHUNK)
    dst_c = dstp.reshape(16, KC, CHUNK)

    xp = jnp.zeros((NP, D_IN), f32).at[:N].set(x)

    deg16 = _deg_kernel(dst_a)
    d0 = deg16[0]
    d1 = deg16[1]

    bn_c = 1.0 / math.sqrt(1.0 + 1e-5)
    s1 = (g1 * bn_c).reshape(1, D_H)
    t1 = (b1 * g1 * bn_c + be1).reshape(1, D_H)
    s2 = (g2 * bn_c).reshape(1, D_H)
    t2 = (b2 * g2 * bn_c + be2).reshape(1, D_H)
    b3r = b3.reshape(1, D_OUT)

    g1lo, g1hi = pl.pallas_call(
        _tc_first_body,
        grid=(GRID,),
        in_specs=[_row_spec(128), _full_spec((128, 128)),
                  _row_spec(16), _row_spec(16)],
        out_specs=[_row_spec(64), _row_spec(64)],
        out_shape=[jax.ShapeDtypeStruct((NP, 64), f32)] * 2,
    )(xp, W1, d0, d1)

    a1lo, a1hi = _edge_kernel_64(g1lo, g1hi, src_c, dst_c)

    g2lo, g2hi = pl.pallas_call(
        functools.partial(_tc_mid_body, dout=128),
        grid=(GRID,),
        in_specs=[_row_spec(64), _row_spec(64), _row_spec(64), _row_spec(64),
                  _row_spec(16), _row_spec(16), _full_spec((128, 128)),
                  _full_spec((1, 128)), _full_spec((1, 128))],
        out_specs=[_row_spec(64), _row_spec(64)],
        out_shape=[jax.ShapeDtypeStruct((NP, 64), f32)] * 2,
    )(a1lo, a1hi, g1lo, g1hi, d0, d1, W2, s1, t1)

    a2lo, a2hi = _edge_kernel_64(g2lo, g2hi, src_c, dst_c)

    g3lo, g3hi = pl.pallas_call(
        functools.partial(_tc_mid_body, dout=64),
        grid=(GRID,),
        in_specs=[_row_spec(64), _row_spec(64), _row_spec(64), _row_spec(64),
                  _row_spec(16), _row_spec(16), _full_spec((128, 64)),
                  _full_spec((1, 128)), _full_spec((1, 128))],
        out_specs=[_row_spec(32), _row_spec(32)],
        out_shape=[jax.ShapeDtypeStruct((NP, 32), f32)] * 2,
    )(a2lo, a2hi, g2lo, g2hi, d0, d1, W3, s2, t2)

    a3lo, a3hi = _edge_kernel_32(g3lo, g3hi, src_c, dst_c)

    out = pl.pallas_call(
        _tc_last_body,
        grid=(GRID,),
        in_specs=[_row_spec(32), _row_spec(32), _row_spec(32), _row_spec(32),
                  _row_spec(16), _row_spec(16), _full_spec((1, 64))],
        out_specs=_row_spec(64),
        out_shape=jax.ShapeDtypeStruct((NP, 64), f32),
    )(a3lo, a3hi, g3lo, g3hi, d0, d1, b3r)

    return out[:N]


# speedup vs baseline: 6.6447x; 6.6447x over previous
import functools
import math

import jax
import jax.numpy as jnp
from jax import lax
from jax.experimental import pallas as pl
from jax.experimental.pallas import tpu as pltpu
from jax.experimental.pallas import tpu_sc as plsc

N = 10000
E = 320000
D_IN = 128
D_H = 128
D_OUT = 64

NP = 10240
CHUNK = 128
KA = 80
E_PAD = 32 * KA * CHUNK
ROWS_PER_TILE = NP // 16

_mesh = plsc.VectorSubcoreMesh(core_axis_name="c", subcore_axis_name="s")


def _fill_rows(ref, nrows, value):
    vv = jnp.full((16,), value, jnp.float32)

    def body(i, _):
        for d in range(8):
            ref[i, pl.ds(16 * d, 16)] = vv
        return 0

    lax.fori_loop(0, nrows, body, 0)


@functools.partial(
    pl.kernel,
    out_type=jax.ShapeDtypeStruct((2, NP, 128), jnp.float32),
    mesh=_mesh,
    scratch_types=[
        pltpu.VMEM((KA, CHUNK), jnp.int32),
        pltpu.VMEM((CHUNK, 128), jnp.float32),
        pltpu.VMEM_SHARED((NP, 128), jnp.float32),
    ],
)
def _deg_kernel(dst3_hbm, deg_hbm, dst_v, buf_v, acc_s):
    cid = lax.axis_index("c")
    sid = lax.axis_index("s")
    wid = cid * 16 + sid

    pltpu.sync_copy(dst3_hbm.at[wid], dst_v)

    _fill_rows(buf_v, CHUNK, 0.0)
    base = sid * ROWS_PER_TILE
    for k in range(ROWS_PER_TILE // CHUNK):
        pltpu.sync_copy(buf_v, acc_s.at[pl.ds(base + k * CHUNK, CHUNK)])
    _fill_rows(buf_v, CHUNK, 1.0)
    plsc.subcore_barrier()

    def edge_chunk(j, _):
        pltpu.sync_copy(buf_v, acc_s.at[dst_v.at[j]], add=True)
        return 0

    lax.fori_loop(0, KA, edge_chunk, 0)
    plsc.subcore_barrier()

    pltpu.sync_copy(
        acc_s.at[pl.ds(base, ROWS_PER_TILE)],
        deg_hbm.at[cid, pl.ds(base, ROWS_PER_TILE)],
    )


@functools.partial(
    pl.kernel,
    out_type=jax.ShapeDtypeStruct((2, NP, 128), jnp.float32),
    mesh=_mesh,
    scratch_types=[
        pltpu.VMEM((KA, CHUNK), jnp.int32),
        pltpu.VMEM((KA, CHUNK), jnp.int32),
        pltpu.VMEM((CHUNK, 128), jnp.float32),
        pltpu.VMEM_SHARED((NP, 128), jnp.float32),
        pltpu.SemaphoreType.DMA,
    ],
)
def _edge_kernel(g_hbm, src3_hbm, dst3_hbm, agg_hbm,
                 src_v, dst_v, rows_v, acc_s, sem):
    cid = lax.axis_index("c")
    sid = lax.axis_index("s")
    wid = cid * 16 + sid

    pltpu.sync_copy(src3_hbm.at[wid], src_v)
    pltpu.sync_copy(dst3_hbm.at[wid], dst_v)

    _fill_rows(rows_v, CHUNK, 0.0)
    base = sid * ROWS_PER_TILE
    for k in range(ROWS_PER_TILE // CHUNK):
        pltpu.sync_copy(rows_v, acc_s.at[pl.ds(base + k * CHUNK, CHUNK)])
    plsc.subcore_barrier()

    def edge_chunk(j, _):
        pltpu.async_copy(g_hbm.at[src_v.at[j]], rows_v, sem).wait()
        pltpu.sync_copy(rows_v, acc_s.at[dst_v.at[j]], add=True)
        return 0

    lax.fori_loop(0, KA, edge_chunk, 0)
    plsc.subcore_barrier()

    pltpu.sync_copy(
        acc_s.at[pl.ds(base, ROWS_PER_TILE)],
        agg_hbm.at[cid, pl.ds(base, ROWS_PER_TILE)],
    )


BLK = 1024
GRID = NP // BLK


def _dinv_blk(d0_ref, d1_ref):
    deg = d0_ref[:, 0] + d1_ref[:, 0] + 1.0
    return 1.0 / jnp.sqrt(deg)


def _tc_first_body(x_ref, w_ref, d0_ref, d1_ref, g_ref):
    dinv = _dinv_blk(d0_ref, d1_ref)
    h = jnp.dot(x_ref[...], w_ref[...], preferred_element_type=jnp.float32)
    g_ref[...] = h * dinv[:, None]


def _tc_mid_body(a0_ref, a1_ref, g_ref, d0_ref, d1_ref,
                 w_ref, s_ref, t_ref, o_ref):
    dinv = _dinv_blk(d0_ref, d1_ref)
    u = (a0_ref[...] + a1_ref[...] + g_ref[...]) * dinv[:, None]
    h = jnp.maximum(u * s_ref[...] + t_ref[...], 0.0)
    o_ref[...] = jnp.dot(
        h, w_ref[...], preferred_element_type=jnp.float32) * dinv[:, None]


def _tc_last_body(a0_ref, a1_ref, g_ref, d0_ref, d1_ref, b_ref, out_ref):
    dinv = _dinv_blk(d0_ref, d1_ref)
    z = ((a0_ref[...] + a1_ref[...] + g_ref[...]) * dinv[:, None])[:, :D_OUT]
    z = z + b_ref[...]
    m = jnp.max(z, axis=1, keepdims=True)
    zs = z - m
    out_ref[...] = zs - jnp.log(jnp.sum(jnp.exp(zs), axis=1, keepdims=True))


def _row_spec(w):
    return pl.BlockSpec((BLK, w), lambda i: (i, 0))


def _full_spec(shape):
    return pl.BlockSpec(shape, lambda i: tuple(0 for _ in shape))


def _mid_call(a2, g, d0, d1, w, s, t):
    return pl.pallas_call(
        _tc_mid_body,
        grid=(GRID,),
        in_specs=[_row_spec(128), _row_spec(128), _row_spec(128),
                  _row_spec(16), _row_spec(16), _full_spec((128, 128)),
                  _full_spec((1, 128)), _full_spec((1, 128))],
        out_specs=_row_spec(128),
        out_shape=jax.ShapeDtypeStruct((NP, 128), jnp.float32),
    )(a2[0], a2[1], g, d0, d1, w, s, t)


def kernel(x, edge_index, W1, b1, g1, be1, W2, b2, g2, be2, W3, b3):
    f32 = jnp.float32
    src = edge_index[0].astype(jnp.int32)
    dst = edge_index[1].astype(jnp.int32)
    pad = jnp.full((E_PAD - E,), N, jnp.int32)
    src3 = jnp.concatenate([src, pad]).reshape(32, KA, CHUNK)
    dst3 = jnp.concatenate([dst, pad]).reshape(32, KA, CHUNK)

    xp = jnp.zeros((NP, D_IN), f32).at[:N].set(x)

    deg2 = _deg_kernel(dst3)
    d0 = deg2[0, :, :16]
    d1 = deg2[1, :, :16]

    bn_c = 1.0 / math.sqrt(1.0 + 1e-5)
    s1 = (g1 * bn_c).reshape(1, D_H)
    t1 = (b1 * g1 * bn_c + be1).reshape(1, D_H)
    s2 = (g2 * bn_c).reshape(1, D_H)
    t2 = (b2 * g2 * bn_c + be2).reshape(1, D_H)
    b3r = b3.reshape(1, D_OUT)
    W3p = jnp.zeros((D_H, 128), f32).at[:, :D_OUT].set(W3)

    ga = pl.pallas_call(
        _tc_first_body,
        grid=(GRID,),
        in_specs=[_row_spec(128), _full_spec((128, 128)),
                  _row_spec(16), _row_spec(16)],
        out_specs=_row_spec(128),
        out_shape=jax.ShapeDtypeStruct((NP, 128), f32),
    )(xp, W1, d0, d1)

    agg1 = _edge_kernel(ga, src3, dst3)
    gb = _mid_call(agg1, ga, d0, d1, W2, s1, t1)
    agg2 = _edge_kernel(gb, src3, dst3)
    gc = _mid_call(agg2, gb, d0, d1, W3p, s2, t2)
    agg3 = _edge_kernel(gc, src3, dst3)

    out = pl.pallas_call(
        _tc_last_body,
        grid=(GRID,),
        in_specs=[_row_spec(128), _row_spec(128), _row_spec(128),
                  _row_spec(16), _row_spec(16), _full_spec((1, 64))],
        out_specs=_row_spec(64),
        out_shape=jax.ShapeDtypeStruct((NP, 64), f32),
    )(agg3[0], agg3[1], gc, d0, d1, b3r)

    return out[:N]
